# Initial kernel scaffold; baseline (speedup 1.0000x reference)
import jax
import jax.numpy as jnp
from jax.experimental import pallas as pl


def kernel(x, edge_index, batch, W1, b1, W2, b2, Wfc, bfc, Wdec, bdec, W3, b3):
    raise NotImplementedError("write your pallas kernel here")


# speedup vs baseline: 7.1521x; 7.1521x over previous
import functools

import jax
import jax.numpy as jnp
from jax import lax
from jax.experimental import pallas as pl
from jax.experimental.pallas import tpu as pltpu
from jax.experimental.pallas import tpu_sc as plsc

NN = 10000
DD = 128
GG = 64
NPAD = 10240
NC, NS = 2, 16
NW = NC * NS
CHUNK = 128
ROWS_T = NPAD // NS
BLK = 1024
NBLK = NPAD // BLK

def _sc_mesh():
    return plsc.VectorSubcoreMesh(core_axis_name="c", subcore_axis_name="s")


IDXB = 8


def _sc_scatter_body(nchunks, table, srcs, dsts, zeros, out,
                     idx_s, idx_d, buf0, buf1, acc, sem0, sem1):
    c = lax.axis_index("c")
    s = lax.axis_index("s")
    w = s * NC + c
    pltpu.sync_copy(zeros, buf0)
    for i in range(ROWS_T // CHUNK):
        pltpu.sync_copy(buf0, acc.at[pl.ds(s * ROWS_T + i * CHUNK, CHUNK)])
    plsc.subcore_barrier()

    def body(blk, carry):
        pltpu.sync_copy(srcs.at[w].at[pl.ds(blk * IDXB, IDXB)], idx_s)
        pltpu.sync_copy(dsts.at[w].at[pl.ds(blk * IDXB, IDXB)], idx_d)
        for j in range(0, IDXB, 2):
            d0 = pltpu.async_copy(table.at[idx_s.at[j]], buf0, sem0)
            d1 = pltpu.async_copy(table.at[idx_s.at[j + 1]], buf1, sem1)
            d0.wait()
            pltpu.sync_copy(buf0, acc.at[idx_d.at[j]], add=True)
            d1.wait()
            pltpu.sync_copy(buf1, acc.at[idx_d.at[j + 1]], add=True)
        return carry

    lax.fori_loop(0, nchunks // IDXB, body, 0)
    plsc.subcore_barrier()
    for i in range(ROWS_T // CHUNK):
        off = s * ROWS_T + i * CHUNK
        pltpu.sync_copy(acc.at[pl.ds(off, CHUNK)], buf0)
        pltpu.sync_copy(buf0, out.at[c].at[pl.ds(off, CHUNK)])


def _sc_deg_body(nchunks, dsts, zeros, ones, out,
                 idx_d, buf0, buf1, acc, sem0, sem1):
    c = lax.axis_index("c")
    s = lax.axis_index("s")
    w = s * NC + c
    pltpu.sync_copy(zeros, buf0)
    for i in range(ROWS_T // CHUNK):
        pltpu.sync_copy(buf0, acc.at[pl.ds(s * ROWS_T + i * CHUNK, CHUNK)])
    plsc.subcore_barrier()
    pltpu.sync_copy(ones, buf1)

    def body(blk, carry):
        pltpu.sync_copy(dsts.at[w].at[pl.ds(blk * IDXB, IDXB)], idx_d)
        for j in range(IDXB):
            pltpu.sync_copy(buf1, acc.at[idx_d.at[j]], add=True)
        return carry

    lax.fori_loop(0, nchunks // IDXB, body, 0)
    plsc.subcore_barrier()
    for i in range(ROWS_T // CHUNK):
        off = s * ROWS_T + i * CHUNK
        pltpu.sync_copy(acc.at[pl.ds(off, CHUNK)], buf0)
        pltpu.sync_copy(buf0, out.at[c].at[pl.ds(off, CHUNK)])


def _sc_scatter(table, src3, dst3, zeros):
    nchunks = src3.shape[1]
    return pl.kernel(
        functools.partial(_sc_scatter_body, nchunks),
        out_type=jax.ShapeDtypeStruct((NC, NPAD, DD), jnp.float32),
        mesh=_sc_mesh(),
        scratch_types=[
            pltpu.VMEM((IDXB, CHUNK), jnp.int32),
            pltpu.VMEM((IDXB, CHUNK), jnp.int32),
            pltpu.VMEM((CHUNK, DD), jnp.float32),
            pltpu.VMEM((CHUNK, DD), jnp.float32),
            pltpu.VMEM_SHARED((NPAD, DD), jnp.float32),
            pltpu.SemaphoreType.DMA,
            pltpu.SemaphoreType.DMA,
        ],
    )(table, src3, dst3, zeros)


def _sc_deg(dst3, zeros, ones):
    nchunks = dst3.shape[1]
    return pl.kernel(
        functools.partial(_sc_deg_body, nchunks),
        out_type=jax.ShapeDtypeStruct((NC, NPAD, DD), jnp.float32),
        mesh=_sc_mesh(),
        scratch_types=[
            pltpu.VMEM((IDXB, CHUNK), jnp.int32),
            pltpu.VMEM((CHUNK, DD), jnp.float32),
            pltpu.VMEM((CHUNK, DD), jnp.float32),
            pltpu.VMEM_SHARED((NPAD, DD), jnp.float32),
            pltpu.SemaphoreType.DMA,
            pltpu.SemaphoreType.DMA,
        ],
    )(dst3, zeros, ones)


def _k1_body(x, w1, p0, p1, hs1, dinvb):
    i = pl.program_id(0)
    deg = p0[...] + p1[...] + 1.0
    rows = lax.broadcasted_iota(jnp.int32, (BLK, DD), 0) + i * BLK
    dv = jnp.where(rows < NN, lax.rsqrt(deg), 0.0)
    hs1[...] = dv * jnp.dot(x[...], w1[...], preferred_element_type=jnp.float32)
    dinvb[...] = dv


def _k2_body(p0, p1, hs, dinvb, b, w, hs_next):
    dv = dinvb[...]
    t = jnp.maximum(dv * (p0[...] + p1[...] + hs[...]) + b[...], 0.0)
    hs_next[...] = dv * jnp.dot(t, w[...], preferred_element_type=jnp.float32)


def _k3_body(p0, p1, hs, dinvb, b, batch, wfc, bfc, wdec, bdec, w3,
             latent, zr, pooled):
    i = pl.program_id(0)
    dv = dinvb[...]
    t2 = jnp.maximum(dv * (p0[...] + p1[...] + hs[...]) + b[...], 0.0)
    bvec = batch[0]
    iota = lax.broadcasted_iota(jnp.int32, (GG, BLK), 0)
    onehot = jnp.where(bvec == iota, 1.0, 0.0)
    contrib = jnp.dot(onehot, t2, preferred_element_type=jnp.float32)

    @pl.when(i == 0)
    def _():
        pooled[...] = contrib

    @pl.when(i > 0)
    def _():
        pooled[...] = pooled[...] + contrib

    @pl.when(i == NBLK - 1)
    def _():
        lat = jnp.maximum(
            jnp.dot(pooled[...], wfc[...], preferred_element_type=jnp.float32)
            + bfc[...], 0.0)
        z = jnp.maximum(
            jnp.dot(lat, wdec[...], preferred_element_type=jnp.float32)
            + bdec[...], 0.0)
        latent[...] = lat
        zr[...] = jnp.dot(z, w3[...], preferred_element_type=jnp.float32)


def _k5_body(batch, dinvb, zr, zh):
    bvec = batch[0]
    iota = lax.broadcasted_iota(jnp.int32, (GG, BLK), 0)
    onehot = jnp.where(bvec == iota, 1.0, 0.0)
    g = lax.dot_general(onehot, zr[...], (((0,), (0,)), ((), ())),
                        preferred_element_type=jnp.float32)
    zh[...] = dinvb[...] * g


def _k6_body(p0, p1, zh, dinvb, b, recon):
    recon[...] = dinvb[...] * (p0[...] + p1[...] + zh[...]) + b[...]


def _row_spec():
    return pl.BlockSpec((BLK, DD), lambda i: (i, 0))


def _full_spec(shape):
    nd = len(shape)
    return pl.BlockSpec(shape, lambda i: (0,) * nd)


def _batch_spec():
    return pl.BlockSpec((1, 1, BLK), lambda i: (i, 0, 0))


def _k1(xpad, W1, p0, p1):
    return pl.pallas_call(
        _k1_body,
        grid=(NBLK,),
        in_specs=[_row_spec(), _full_spec((DD, DD)), _row_spec(), _row_spec()],
        out_specs=[_row_spec(), _row_spec()],
        out_shape=[jax.ShapeDtypeStruct((NPAD, DD), jnp.float32)] * 2,
    )(xpad, W1, p0, p1)


def _k2(p0, p1, hs, dinvb, b, W):
    return pl.pallas_call(
        _k2_body,
        grid=(NBLK,),
        in_specs=[_row_spec(), _row_spec(), _row_spec(), _row_spec(),
                  _full_spec((1, DD)), _full_spec((DD, DD))],
        out_specs=_row_spec(),
        out_shape=jax.ShapeDtypeStruct((NPAD, DD), jnp.float32),
    )(p0, p1, hs, dinvb, b, W)


def _k3(p0, p1, hs, dinvb, b2, batch3, Wfc, bfc, Wdec, bdec, W3):
    return pl.pallas_call(
        _k3_body,
        grid=(NBLK,),
        in_specs=[_row_spec(), _row_spec(), _row_spec(), _row_spec(),
                  _full_spec((1, DD)), _batch_spec(),
                  _full_spec((DD, GG)), _full_spec((1, GG)),
                  _full_spec((GG, DD)), _full_spec((1, DD)),
                  _full_spec((DD, DD))],
        out_specs=[pl.BlockSpec((GG, GG), lambda i: (0, 0)),
                   pl.BlockSpec((GG, DD), lambda i: (0, 0))],
        out_shape=[jax.ShapeDtypeStruct((GG, GG), jnp.float32),
                   jax.ShapeDtypeStruct((GG, DD), jnp.float32)],
        scratch_shapes=[pltpu.VMEM((GG, DD), jnp.float32)],
    )(p0, p1, hs, dinvb, b2, batch3, Wfc, bfc, Wdec, bdec, W3)


def _k5(batch3, dinvb, zr):
    return pl.pallas_call(
        _k5_body,
        grid=(NBLK,),
        in_specs=[_batch_spec(), _row_spec(), _full_spec((GG, DD))],
        out_specs=_row_spec(),
        out_shape=jax.ShapeDtypeStruct((NPAD, DD), jnp.float32),
    )(batch3, dinvb, zr)


def _k6(p0, p1, zh, dinvb, b3):
    return pl.pallas_call(
        _k6_body,
        grid=(NBLK,),
        in_specs=[_row_spec(), _row_spec(), _row_spec(), _row_spec(),
                  _full_spec((1, DD))],
        out_specs=_row_spec(),
        out_shape=jax.ShapeDtypeStruct((NPAD, DD), jnp.float32),
    )(p0, p1, zh, dinvb, b3)


def kernel(x, edge_index, batch, W1, b1, W2, b2, Wfc, bfc, Wdec, bdec, W3, b3):
    e = edge_index.shape[1]
    nchunks = -(-e // (NW * CHUNK))
    nchunks = -(-nchunks // IDXB) * IDXB
    epad = NW * nchunks * CHUNK

    ei = edge_index.astype(jnp.int32)
    fill = jnp.full((epad - e,), NN, jnp.int32)
    src3 = jnp.concatenate([ei[0], fill]).reshape(NW, nchunks, CHUNK)
    dst3 = jnp.concatenate([ei[1], fill]).reshape(NW, nchunks, CHUNK)
    batch3 = jnp.concatenate(
        [batch.astype(jnp.int32), jnp.full((NPAD - NN,), GG, jnp.int32)]
    ).reshape(NBLK, 1, BLK)
    xpad = jnp.pad(x, ((0, NPAD - NN), (0, 0)))
    zeros = jnp.zeros((CHUNK, DD), jnp.float32)
    ones = jnp.ones((CHUNK, DD), jnp.float32)

    b1r = b1.reshape(1, DD)
    b2r = b2.reshape(1, DD)
    b3r = b3.reshape(1, DD)
    bfcr = bfc.reshape(1, GG)
    bdecr = bdec.reshape(1, DD)

    degp = _sc_deg(dst3, zeros, ones)
    hs1, dinvb = _k1(xpad, W1, degp[0], degp[1])
    p1 = _sc_scatter(hs1, src3, dst3, zeros)
    hs2 = _k2(p1[0], p1[1], hs1, dinvb, b1r, W2)
    p2 = _sc_scatter(hs2, src3, dst3, zeros)
    latent, zr = _k3(p2[0], p2[1], hs2, dinvb, b2r, batch3,
                     Wfc, bfcr, Wdec, bdecr, W3)
    zh = _k5(batch3, dinvb, zr)
    p3 = _sc_scatter(zh, src3, dst3, zeros)
    recon = _k6(p3[0], p3[1], zh, dinvb, b3r)
    return (recon[:NN], latent)


# speedup vs baseline: 12.2523x; 1.7131x over previous
import functools

import jax
import jax.numpy as jnp
from jax import lax
from jax.experimental import pallas as pl
from jax.experimental.pallas import tpu as pltpu
from jax.experimental.pallas import tpu_sc as plsc

NN = 10000
DD = 128
GG = 64
NPAD = 10240
NC, NS = 2, 16
NW = NC * NS
CHUNK = 128
ROWS_T = NPAD // NS
BLK = 1024
NBLK = NPAD // BLK

def _sc_mesh():
    return plsc.VectorSubcoreMesh(core_axis_name="c", subcore_axis_name="s")


IDXB = 8

C0_T = 24
C1_T = 136


def _core_chunks(c_t0, c_t1):
    return c_t0, c_t1


def _sc_scatter_body(table, srcs, dsts, zeros, out,
                     idx_s, idx_d, buf0, buf1, acc, sem0, sem1):
    c = lax.axis_index("c")
    s = lax.axis_index("s")
    pltpu.sync_copy(zeros, buf0)
    for i in range(ROWS_T // CHUNK):
        pltpu.sync_copy(buf0, acc.at[pl.ds(s * ROWS_T + i * CHUNK, CHUNK)])
    plsc.subcore_barrier()

    def run(base, count):
        def body(blk, carry):
            row = base + blk * IDXB
            pltpu.sync_copy(srcs.at[pl.ds(row, IDXB)], idx_s)
            pltpu.sync_copy(dsts.at[pl.ds(row, IDXB)], idx_d)
            for j in range(0, IDXB, 2):
                d0 = pltpu.async_copy(table.at[idx_s.at[j]], buf0, sem0)
                d1 = pltpu.async_copy(table.at[idx_s.at[j + 1]], buf1, sem1)
                d0.wait()
                pltpu.sync_copy(buf0, acc.at[idx_d.at[j]], add=True)
                d1.wait()
                pltpu.sync_copy(buf1, acc.at[idx_d.at[j + 1]], add=True)
            return carry
        lax.fori_loop(0, count // IDXB, body, 0)

    @pl.when(c == 0)
    def _():
        run(s * C0_T, C0_T)

    @pl.when(c == 1)
    def _():
        run(NS * C0_T + s * C1_T, C1_T)

    plsc.subcore_barrier()
    for i in range(ROWS_T // CHUNK):
        off = s * ROWS_T + i * CHUNK
        pltpu.sync_copy(acc.at[pl.ds(off, CHUNK)], buf0)
        pltpu.sync_copy(buf0, out.at[c].at[pl.ds(off, CHUNK)])


def _sc_deg_body(dsts, zeros, ones, out,
                 idx_d, buf0, buf1, acc, sem0, sem1):
    c = lax.axis_index("c")
    s = lax.axis_index("s")
    pltpu.sync_copy(zeros, buf0)
    for i in range(ROWS_T // CHUNK):
        pltpu.sync_copy(buf0, acc.at[pl.ds(s * ROWS_T + i * CHUNK, CHUNK)])
    plsc.subcore_barrier()
    pltpu.sync_copy(ones, buf1)

    def run(base, count):
        def body(blk, carry):
            pltpu.sync_copy(dsts.at[pl.ds(base + blk * IDXB, IDXB)], idx_d)
            for j in range(IDXB):
                pltpu.sync_copy(buf1, acc.at[idx_d.at[j]], add=True)
            return carry
        lax.fori_loop(0, count // IDXB, body, 0)

    half = (C0_T + C1_T) * NS // 2
    even_t = half // NS - (half // NS) % IDXB
    rem_t = (C0_T + C1_T) - even_t

    @pl.when(c == 0)
    def _():
        run(s * even_t, even_t)

    @pl.when(c == 1)
    def _():
        run(NS * even_t + s * rem_t, rem_t)

    plsc.subcore_barrier()
    for i in range(ROWS_T // CHUNK):
        off = s * ROWS_T + i * CHUNK
        pltpu.sync_copy(acc.at[pl.ds(off, CHUNK)], buf0)
        pltpu.sync_copy(buf0, out.at[c].at[pl.ds(off, CHUNK)])


def _sc_scatter(table, src2, dst2, zeros):
    return pl.kernel(
        _sc_scatter_body,
        out_type=jax.ShapeDtypeStruct((NC, NPAD, DD), jnp.float32),
        mesh=_sc_mesh(),
        scratch_types=[
            pltpu.VMEM((IDXB, CHUNK), jnp.int32),
            pltpu.VMEM((IDXB, CHUNK), jnp.int32),
            pltpu.VMEM((CHUNK, DD), jnp.float32),
            pltpu.VMEM((CHUNK, DD), jnp.float32),
            pltpu.VMEM_SHARED((NPAD, DD), jnp.float32),
            pltpu.SemaphoreType.DMA,
            pltpu.SemaphoreType.DMA,
        ],
    )(table, src2, dst2, zeros)


def _sc_deg(dst2, zeros, ones):
    return pl.kernel(
        _sc_deg_body,
        out_type=jax.ShapeDtypeStruct((NC, NPAD, DD), jnp.float32),
        mesh=_sc_mesh(),
        scratch_types=[
            pltpu.VMEM((IDXB, CHUNK), jnp.int32),
            pltpu.VMEM((CHUNK, DD), jnp.float32),
            pltpu.VMEM((CHUNK, DD), jnp.float32),
            pltpu.VMEM_SHARED((NPAD, DD), jnp.float32),
            pltpu.SemaphoreType.DMA,
            pltpu.SemaphoreType.DMA,
        ],
    )(dst2, zeros, ones)


def _k1_body(x, w1, p0, p1, hs1, dinvb):
    i = pl.program_id(0)
    deg = p0[...] + p1[...] + 1.0
    rows = lax.broadcasted_iota(jnp.int32, (BLK, DD), 0) + i * BLK
    dv = jnp.where(rows < NN, lax.rsqrt(deg), 0.0)
    hs1[...] = dv * jnp.dot(x[...], w1[...], preferred_element_type=jnp.float32)
    dinvb[...] = dv


def _k2_body(p0, p1, hs, dinvb, b, w, hs_next):
    dv = dinvb[...]
    t = jnp.maximum(dv * (p0[...] + p1[...] + hs[...]) + b[...], 0.0)
    hs_next[...] = dv * jnp.dot(t, w[...], preferred_element_type=jnp.float32)


def _k3_body(p0, p1, hs, dinvb, b, batch, wfc, bfc, wdec, bdec, w3,
             latent, zr, pooled):
    i = pl.program_id(0)
    dv = dinvb[...]
    t2 = jnp.maximum(dv * (p0[...] + p1[...] + hs[...]) + b[...], 0.0)
    bvec = batch[0]
    iota = lax.broadcasted_iota(jnp.int32, (GG, BLK), 0)
    onehot = jnp.where(bvec == iota, 1.0, 0.0)
    contrib = jnp.dot(onehot, t2, preferred_element_type=jnp.float32)

    @pl.when(i == 0)
    def _():
        pooled[...] = contrib

    @pl.when(i > 0)
    def _():
        pooled[...] = pooled[...] + contrib

    @pl.when(i == NBLK - 1)
    def _():
        lat = jnp.maximum(
            jnp.dot(pooled[...], wfc[...], preferred_element_type=jnp.float32)
            + bfc[...], 0.0)
        z = jnp.maximum(
            jnp.dot(lat, wdec[...], preferred_element_type=jnp.float32)
            + bdec[...], 0.0)
        latent[...] = lat
        zr[...] = jnp.dot(z, w3[...], preferred_element_type=jnp.float32)


def _k5_body(batch, dinvb, zr, zh):
    bvec = batch[0]
    iota = lax.broadcasted_iota(jnp.int32, (GG, BLK), 0)
    onehot = jnp.where(bvec == iota, 1.0, 0.0)
    g = lax.dot_general(onehot, zr[...], (((0,), (0,)), ((), ())),
                        preferred_element_type=jnp.float32)
    zh[...] = dinvb[...] * g


def _k6_body(p0, p1, zh, dinvb, b, recon):
    recon[...] = dinvb[...] * (p0[...] + p1[...] + zh[...]) + b[...]


def _row_spec():
    return pl.BlockSpec((BLK, DD), lambda i: (i, 0))


def _full_spec(shape):
    nd = len(shape)
    return pl.BlockSpec(shape, lambda i: (0,) * nd)


def _batch_spec():
    return pl.BlockSpec((1, 1, BLK), lambda i: (i, 0, 0))


def _k1(xpad, W1, p0, p1):
    return pl.pallas_call(
        _k1_body,
        grid=(NBLK,),
        in_specs=[_row_spec(), _full_spec((DD, DD)), _row_spec(), _row_spec()],
        out_specs=[_row_spec(), _row_spec()],
        out_shape=[jax.ShapeDtypeStruct((NPAD, DD), jnp.float32)] * 2,
    )(xpad, W1, p0, p1)


def _k2(p0, p1, hs, dinvb, b, W):
    return pl.pallas_call(
        _k2_body,
        grid=(NBLK,),
        in_specs=[_row_spec(), _row_spec(), _row_spec(), _row_spec(),
                  _full_spec((1, DD)), _full_spec((DD, DD))],
        out_specs=_row_spec(),
        out_shape=jax.ShapeDtypeStruct((NPAD, DD), jnp.float32),
    )(p0, p1, hs, dinvb, b, W)


def _k3(p0, p1, hs, dinvb, b2, batch3, Wfc, bfc, Wdec, bdec, W3):
    return pl.pallas_call(
        _k3_body,
        grid=(NBLK,),
        in_specs=[_row_spec(), _row_spec(), _row_spec(), _row_spec(),
                  _full_spec((1, DD)), _batch_spec(),
                  _full_spec((DD, GG)), _full_spec((1, GG)),
                  _full_spec((GG, DD)), _full_spec((1, DD)),
                  _full_spec((DD, DD))],
        out_specs=[pl.BlockSpec((GG, GG), lambda i: (0, 0)),
                   pl.BlockSpec((GG, DD), lambda i: (0, 0))],
        out_shape=[jax.ShapeDtypeStruct((GG, GG), jnp.float32),
                   jax.ShapeDtypeStruct((GG, DD), jnp.float32)],
        scratch_shapes=[pltpu.VMEM((GG, DD), jnp.float32)],
    )(p0, p1, hs, dinvb, b2, batch3, Wfc, bfc, Wdec, bdec, W3)


def _k5(batch3, dinvb, zr):
    return pl.pallas_call(
        _k5_body,
        grid=(NBLK,),
        in_specs=[_batch_spec(), _row_spec(), _full_spec((GG, DD))],
        out_specs=_row_spec(),
        out_shape=jax.ShapeDtypeStruct((NPAD, DD), jnp.float32),
    )(batch3, dinvb, zr)


def _k6(p0, p1, zh, dinvb, b3):
    return pl.pallas_call(
        _k6_body,
        grid=(NBLK,),
        in_specs=[_row_spec(), _row_spec(), _row_spec(), _row_spec(),
                  _full_spec((1, DD))],
        out_specs=_row_spec(),
        out_shape=jax.ShapeDtypeStruct((NPAD, DD), jnp.float32),
    )(p0, p1, zh, dinvb, b3)


def kernel(x, edge_index, batch, W1, b1, W2, b2, Wfc, bfc, Wdec, bdec, W3, b3):
    e = edge_index.shape[1]
    cap = NS * (C0_T + C1_T)
    epad = cap * CHUNK
    assert epad >= e

    ei = edge_index.astype(jnp.int32)
    fill = NN + (jnp.arange(epad - e, dtype=jnp.int32) % (NPAD - NN))
    src3 = jnp.concatenate([ei[0], fill]).reshape(cap, CHUNK)
    dst3 = jnp.concatenate([ei[1], fill]).reshape(cap, CHUNK)
    batch3 = jnp.concatenate(
        [batch.astype(jnp.int32), jnp.full((NPAD - NN,), GG, jnp.int32)]
    ).reshape(NBLK, 1, BLK)
    xpad = jnp.pad(x, ((0, NPAD - NN), (0, 0)))
    zeros = jnp.zeros((CHUNK, DD), jnp.float32)
    ones = jnp.ones((CHUNK, DD), jnp.float32)

    b1r = b1.reshape(1, DD)
    b2r = b2.reshape(1, DD)
    b3r = b3.reshape(1, DD)
    bfcr = bfc.reshape(1, GG)
    bdecr = bdec.reshape(1, DD)

    degp = _sc_deg(dst3, zeros, ones)
    hs1, dinvb = _k1(xpad, W1, degp[0], degp[1])
    p1 = _sc_scatter(hs1, src3, dst3, zeros)
    hs2 = _k2(p1[0], p1[1], hs1, dinvb, b1r, W2)
    p2 = _sc_scatter(hs2, src3, dst3, zeros)
    latent, zr = _k3(p2[0], p2[1], hs2, dinvb, b2r, batch3,
                     Wfc, bfcr, Wdec, bdecr, W3)
    zh = _k5(batch3, dinvb, zr)
    p3 = _sc_scatter(zh, src3, dst3, zeros)
    recon = _k6(p3[0], p3[1], zh, dinvb, b3r)
    return (recon[:NN], latent)


# speedup vs baseline: 18.0307x; 1.4716x over previous
import functools

import jax
import jax.numpy as jnp
from jax import lax
from jax.experimental import pallas as pl
from jax.experimental.pallas import tpu as pltpu
from jax.experimental.pallas import tpu_sc as plsc

NN = 10000
DD = 128
GG = 64
NPAD = 10240
NC, NS = 2, 16
NW = NC * NS
CHUNK = 128
ROWS_T = NPAD // NS
BLK = 1024
NBLK = NPAD // BLK

def _sc_mesh():
    return plsc.VectorSubcoreMesh(core_axis_name="c", subcore_axis_name="s")


IDXB = 8

C0_T = 80
C1_T = 80


def _core_chunks(c_t0, c_t1):
    return c_t0, c_t1


def _sc_scatter_body(table, srcs, dsts, zeros, out,
                     idx_s, idx_d, buf0, buf1, acc, sem0, sem1):
    c = lax.axis_index("c")
    s = lax.axis_index("s")
    pltpu.sync_copy(zeros, buf0)
    for i in range(ROWS_T // CHUNK):
        pltpu.sync_copy(buf0, acc.at[pl.ds(s * ROWS_T + i * CHUNK, CHUNK)])
    plsc.subcore_barrier()

    def run(base, count):
        def body(blk, carry):
            row = base + blk * IDXB
            pltpu.sync_copy(srcs.at[pl.ds(row, IDXB)], idx_s)
            pltpu.sync_copy(dsts.at[pl.ds(row, IDXB)], idx_d)
            for j in range(0, IDXB, 2):
                d0 = pltpu.async_copy(table.at[idx_s.at[j]], buf0, sem0)
                d1 = pltpu.async_copy(table.at[idx_s.at[j + 1]], buf1, sem1)
                d0.wait()
                pltpu.sync_copy(buf0, acc.at[idx_d.at[j]], add=True)
                d1.wait()
                pltpu.sync_copy(buf1, acc.at[idx_d.at[j + 1]], add=True)
            return carry
        lax.fori_loop(0, count // IDXB, body, 0)

    @pl.when(c == 0)
    def _():
        run(s * C0_T, C0_T)

    @pl.when(c == 1)
    def _():
        run(NS * C0_T + s * C1_T, C1_T)

    plsc.subcore_barrier()
    for i in range(ROWS_T // CHUNK):
        off = s * ROWS_T + i * CHUNK
        pltpu.sync_copy(acc.at[pl.ds(off, CHUNK)], buf0)
        pltpu.sync_copy(buf0, out.at[c].at[pl.ds(off, CHUNK)])


def _sc_deg_body(dsts, zeros, ones, out,
                 idx_d, buf0, buf1, acc, sem0, sem1):
    c = lax.axis_index("c")
    s = lax.axis_index("s")
    pltpu.sync_copy(zeros, buf0)
    for i in range(ROWS_T // CHUNK):
        pltpu.sync_copy(buf0, acc.at[pl.ds(s * ROWS_T + i * CHUNK, CHUNK)])
    plsc.subcore_barrier()
    pltpu.sync_copy(ones, buf1)

    def run(base, count):
        def body(blk, carry):
            pltpu.sync_copy(dsts.at[pl.ds(base + blk * IDXB, IDXB)], idx_d)
            for j in range(IDXB):
                pltpu.sync_copy(buf1, acc.at[idx_d.at[j]], add=True)
            return carry
        lax.fori_loop(0, count // IDXB, body, 0)

    half = (C0_T + C1_T) * NS // 2
    even_t = half // NS - (half // NS) % IDXB
    rem_t = (C0_T + C1_T) - even_t

    @pl.when(c == 0)
    def _():
        run(s * even_t, even_t)

    @pl.when(c == 1)
    def _():
        run(NS * even_t + s * rem_t, rem_t)

    plsc.subcore_barrier()
    for i in range(ROWS_T // CHUNK):
        off = s * ROWS_T + i * CHUNK
        pltpu.sync_copy(acc.at[pl.ds(off, CHUNK)], buf0)
        pltpu.sync_copy(buf0, out.at[c].at[pl.ds(off, CHUNK)])


def _sc_scatter(table, src2, dst2, zeros):
    return pl.kernel(
        _sc_scatter_body,
        out_type=jax.ShapeDtypeStruct((NC, NPAD, DD), jnp.float32),
        mesh=_sc_mesh(),
        scratch_types=[
            pltpu.VMEM((IDXB, CHUNK), jnp.int32),
            pltpu.VMEM((IDXB, CHUNK), jnp.int32),
            pltpu.VMEM((CHUNK, DD), jnp.float32),
            pltpu.VMEM((CHUNK, DD), jnp.float32),
            pltpu.VMEM_SHARED((NPAD, DD), jnp.float32),
            pltpu.SemaphoreType.DMA,
            pltpu.SemaphoreType.DMA,
        ],
    )(table, src2, dst2, zeros)


def _sc_deg(dst2, zeros, ones):
    return pl.kernel(
        _sc_deg_body,
        out_type=jax.ShapeDtypeStruct((NC, NPAD, DD), jnp.float32),
        mesh=_sc_mesh(),
        scratch_types=[
            pltpu.VMEM((IDXB, CHUNK), jnp.int32),
            pltpu.VMEM((CHUNK, DD), jnp.float32),
            pltpu.VMEM((CHUNK, DD), jnp.float32),
            pltpu.VMEM_SHARED((NPAD, DD), jnp.float32),
            pltpu.SemaphoreType.DMA,
            pltpu.SemaphoreType.DMA,
        ],
    )(dst2, zeros, ones)


def _k0_body(x, w1, h1):
    h1[...] = jnp.dot(x[...], w1[...], preferred_element_type=jnp.float32)


def _k1_body(h1, parts, hs1, dinvb):
    i = pl.program_id(0)
    deg = parts[0] + parts[1] + 1.0
    rows = lax.broadcasted_iota(jnp.int32, (BLK, DD), 0) + i * BLK
    dv = jnp.where(rows < NN, lax.rsqrt(deg), 0.0)
    hs1[...] = dv * h1[...]
    dinvb[...] = dv


def _k2_body(parts, hs, dinvb, b, w, hs_next):
    dv = dinvb[...]
    t = jnp.maximum(dv * (parts[0] + parts[1] + hs[...]) + b[...], 0.0)
    hs_next[...] = dv * jnp.dot(t, w[...], preferred_element_type=jnp.float32)


def _k3_body(parts, hs, dinvb, b, batch, wfc, bfc, wdec, bdec, w3,
             latent, zr, pooled):
    i = pl.program_id(0)
    dv = dinvb[...]
    t2 = jnp.maximum(dv * (parts[0] + parts[1] + hs[...]) + b[...], 0.0)
    bvec = batch[0]
    iota = lax.broadcasted_iota(jnp.int32, (GG, BLK), 0)
    onehot = jnp.where(bvec == iota, 1.0, 0.0)
    contrib = jnp.dot(onehot, t2, preferred_element_type=jnp.float32)

    @pl.when(i == 0)
    def _():
        pooled[...] = contrib

    @pl.when(i > 0)
    def _():
        pooled[...] = pooled[...] + contrib

    @pl.when(i == NBLK - 1)
    def _():
        lat = jnp.maximum(
            jnp.dot(pooled[...], wfc[...], preferred_element_type=jnp.float32)
            + bfc[...], 0.0)
        z = jnp.maximum(
            jnp.dot(lat, wdec[...], preferred_element_type=jnp.float32)
            + bdec[...], 0.0)
        latent[...] = lat
        zr[...] = jnp.dot(z, w3[...], preferred_element_type=jnp.float32)


def _k5_body(batch, dinvb, zr, zh):
    bvec = batch[0]
    iota = lax.broadcasted_iota(jnp.int32, (GG, BLK), 0)
    onehot = jnp.where(bvec == iota, 1.0, 0.0)
    g = lax.dot_general(onehot, zr[...], (((0,), (0,)), ((), ())),
                        preferred_element_type=jnp.float32)
    zh[...] = dinvb[...] * g


def _k6_body(parts, zh, dinvb, b, recon):
    recon[...] = dinvb[...] * (parts[0] + parts[1] + zh[...]) + b[...]


def _row_spec():
    return pl.BlockSpec((BLK, DD), lambda i: (i, 0))


def _parts_spec():
    return pl.BlockSpec((NC, BLK, DD), lambda i: (0, i, 0))


def _full_spec(shape):
    nd = len(shape)
    return pl.BlockSpec(shape, lambda i: (0,) * nd)


def _batch_spec():
    return pl.BlockSpec((1, 1, BLK), lambda i: (i, 0, 0))


def _k0(xpad, W1):
    return pl.pallas_call(
        _k0_body,
        grid=(NBLK,),
        in_specs=[_row_spec(), _full_spec((DD, DD))],
        out_specs=_row_spec(),
        out_shape=jax.ShapeDtypeStruct((NPAD, DD), jnp.float32),
    )(xpad, W1)


def _k1(h1, parts):
    return pl.pallas_call(
        _k1_body,
        grid=(NBLK,),
        in_specs=[_row_spec(), _parts_spec()],
        out_specs=[_row_spec(), _row_spec()],
        out_shape=[jax.ShapeDtypeStruct((NPAD, DD), jnp.float32)] * 2,
    )(h1, parts)


def _k2(parts, hs, dinvb, b, W):
    return pl.pallas_call(
        _k2_body,
        grid=(NBLK,),
        in_specs=[_parts_spec(), _row_spec(), _row_spec(),
                  _full_spec((1, DD)), _full_spec((DD, DD))],
        out_specs=_row_spec(),
        out_shape=jax.ShapeDtypeStruct((NPAD, DD), jnp.float32),
    )(parts, hs, dinvb, b, W)


def _k3(parts, hs, dinvb, b2, batch3, Wfc, bfc, Wdec, bdec, W3):
    return pl.pallas_call(
        _k3_body,
        grid=(NBLK,),
        in_specs=[_parts_spec(), _row_spec(), _row_spec(),
                  _full_spec((1, DD)), _batch_spec(),
                  _full_spec((DD, GG)), _full_spec((1, GG)),
                  _full_spec((GG, DD)), _full_spec((1, DD)),
                  _full_spec((DD, DD))],
        out_specs=[pl.BlockSpec((GG, GG), lambda i: (0, 0)),
                   pl.BlockSpec((GG, DD), lambda i: (0, 0))],
        out_shape=[jax.ShapeDtypeStruct((GG, GG), jnp.float32),
                   jax.ShapeDtypeStruct((GG, DD), jnp.float32)],
        scratch_shapes=[pltpu.VMEM((GG, DD), jnp.float32)],
    )(parts, hs, dinvb, b2, batch3, Wfc, bfc, Wdec, bdec, W3)


def _k5(batch3, dinvb, zr):
    return pl.pallas_call(
        _k5_body,
        grid=(NBLK,),
        in_specs=[_batch_spec(), _row_spec(), _full_spec((GG, DD))],
        out_specs=_row_spec(),
        out_shape=jax.ShapeDtypeStruct((NPAD, DD), jnp.float32),
    )(batch3, dinvb, zr)


def _k6(parts, zh, dinvb, b3):
    return pl.pallas_call(
        _k6_body,
        grid=(NBLK,),
        in_specs=[_parts_spec(), _row_spec(), _row_spec(),
                  _full_spec((1, DD))],
        out_specs=_row_spec(),
        out_shape=jax.ShapeDtypeStruct((NPAD, DD), jnp.float32),
    )(parts, zh, dinvb, b3)


def kernel(x, edge_index, batch, W1, b1, W2, b2, Wfc, bfc, Wdec, bdec, W3, b3):
    e = edge_index.shape[1]
    cap = NS * (C0_T + C1_T)
    epad = cap * CHUNK
    assert epad >= e

    ei = edge_index.astype(jnp.int32)
    fill = NN + (jnp.arange(epad - e, dtype=jnp.int32) % (NPAD - NN))
    src3 = jnp.concatenate([ei[0], fill]).reshape(cap, CHUNK)
    dst3 = jnp.concatenate([ei[1], fill]).reshape(cap, CHUNK)
    batch3 = jnp.concatenate(
        [batch.astype(jnp.int32), jnp.full((NPAD - NN,), GG, jnp.int32)]
    ).reshape(NBLK, 1, BLK)
    xpad = jnp.pad(x, ((0, NPAD - NN), (0, 0)))
    zeros = jnp.zeros((CHUNK, DD), jnp.float32)
    ones = jnp.ones((CHUNK, DD), jnp.float32)

    b1r = b1.reshape(1, DD)
    b2r = b2.reshape(1, DD)
    b3r = b3.reshape(1, DD)
    bfcr = bfc.reshape(1, GG)
    bdecr = bdec.reshape(1, DD)

    degp = _sc_deg(dst3, zeros, ones)
    h1 = _k0(xpad, W1)
    hs1, dinvb = _k1(h1, degp)
    p1 = _sc_scatter(hs1, src3, dst3, zeros)
    hs2 = _k2(p1, hs1, dinvb, b1r, W2)
    p2 = _sc_scatter(hs2, src3, dst3, zeros)
    latent, zr = _k3(p2, hs2, dinvb, b2r, batch3,
                     Wfc, bfcr, Wdec, bdecr, W3)
    zh = _k5(batch3, dinvb, zr)
    p3 = _sc_scatter(zh, src3, dst3, zeros)
    recon = _k6(p3, zh, dinvb, b3r)
    return (recon[:NN], latent)


# speedup vs baseline: 23.0101x; 1.2762x over previous
import functools

import jax
import jax.numpy as jnp
from jax import lax
from jax.experimental import pallas as pl
from jax.experimental.pallas import tpu as pltpu
from jax.experimental.pallas import tpu_sc as plsc

NN = 10000
DD = 128
GG = 64
NPAD = 10240
NC, NS = 2, 16
NW = NC * NS
CHUNK = 128
ROWS_T = NPAD // NS
BLK = 1024
NBLK = NPAD // BLK

def _sc_mesh():
    return plsc.VectorSubcoreMesh(core_axis_name="c", subcore_axis_name="s")


IDXB = 8

C0_T = 80
C1_T = 80


def _core_chunks(c_t0, c_t1):
    return c_t0, c_t1


def _sc_scatter_body(table, srcs, dsts, zeros, out,
                     idx_s, idx_d, buf0, buf1, acc, sem0, sem1, sem2, sem3):
    c = lax.axis_index("c")
    s = lax.axis_index("s")
    pltpu.sync_copy(zeros, buf0)
    for i in range(ROWS_T // CHUNK):
        pltpu.sync_copy(buf0, acc.at[pl.ds(s * ROWS_T + i * CHUNK, CHUNK)])
    plsc.subcore_barrier()

    bufs = (buf0, buf1)
    gsems = (sem0, sem1)
    ssems = (sem2, sem3)

    def run(base, count):
        def body(blk, carry):
            row = base + blk * IDXB
            pltpu.sync_copy(srcs.at[pl.ds(row, IDXB)], idx_s)
            pltpu.sync_copy(dsts.at[pl.ds(row, IDXB)], idx_d)
            ds = [None] * IDXB
            ss = [None] * IDXB
            ds[0] = pltpu.async_copy(table.at[idx_s.at[0]], buf0, gsems[0])
            ds[1] = pltpu.async_copy(table.at[idx_s.at[1]], buf1, gsems[1])
            for j in range(IDXB):
                b = j % 2
                ds[j].wait()
                ss[j] = pltpu.async_copy(bufs[b], acc.at[idx_d.at[j]],
                                         ssems[b], add=True)
                if j + 2 < IDXB:
                    ss[j].wait()
                    ds[j + 2] = pltpu.async_copy(table.at[idx_s.at[j + 2]],
                                                 bufs[b], gsems[b])
            ss[IDXB - 2].wait()
            ss[IDXB - 1].wait()
            return carry
        lax.fori_loop(0, count // IDXB, body, 0)

    @pl.when(c == 0)
    def _():
        run(s * C0_T, C0_T)

    @pl.when(c == 1)
    def _():
        run(NS * C0_T + s * C1_T, C1_T)

    plsc.subcore_barrier()
    for i in range(ROWS_T // CHUNK):
        off = s * ROWS_T + i * CHUNK
        pltpu.sync_copy(acc.at[pl.ds(off, CHUNK)], buf0)
        pltpu.sync_copy(buf0, out.at[c].at[pl.ds(off, CHUNK)])


def _sc_deg_body(dsts, zeros, ones, out,
                 idx_d, buf0, buf1, acc, sem0, sem1):
    c = lax.axis_index("c")
    s = lax.axis_index("s")
    pltpu.sync_copy(zeros, buf0)
    for i in range(ROWS_T // CHUNK):
        pltpu.sync_copy(buf0, acc.at[pl.ds(s * ROWS_T + i * CHUNK, CHUNK)])
    plsc.subcore_barrier()
    pltpu.sync_copy(ones, buf1)

    def run(base, count):
        def body(blk, carry):
            pltpu.sync_copy(dsts.at[pl.ds(base + blk * IDXB, IDXB)], idx_d)
            ss = [pltpu.async_copy(buf1, acc.at[idx_d.at[j]], sem1, add=True)
                  for j in range(IDXB)]
            for d in ss:
                d.wait()
            return carry
        lax.fori_loop(0, count // IDXB, body, 0)

    half = (C0_T + C1_T) * NS // 2
    even_t = half // NS - (half // NS) % IDXB
    rem_t = (C0_T + C1_T) - even_t

    @pl.when(c == 0)
    def _():
        run(s * even_t, even_t)

    @pl.when(c == 1)
    def _():
        run(NS * even_t + s * rem_t, rem_t)

    plsc.subcore_barrier()
    for i in range(ROWS_T // CHUNK):
        off = s * ROWS_T + i * CHUNK
        pltpu.sync_copy(acc.at[pl.ds(off, CHUNK)], buf0)
        pltpu.sync_copy(buf0, out.at[c].at[pl.ds(off, CHUNK)])


def _sc_scatter(table, src2, dst2, zeros):
    return pl.kernel(
        _sc_scatter_body,
        out_type=jax.ShapeDtypeStruct((NC, NPAD, DD), jnp.float32),
        mesh=_sc_mesh(),
        scratch_types=[
            pltpu.VMEM((IDXB, CHUNK), jnp.int32),
            pltpu.VMEM((IDXB, CHUNK), jnp.int32),
            pltpu.VMEM((CHUNK, DD), jnp.float32),
            pltpu.VMEM((CHUNK, DD), jnp.float32),
            pltpu.VMEM_SHARED((NPAD, DD), jnp.float32),
            pltpu.SemaphoreType.DMA,
            pltpu.SemaphoreType.DMA,
            pltpu.SemaphoreType.DMA,
            pltpu.SemaphoreType.DMA,
        ],
    )(table, src2, dst2, zeros)


DEGW = 16


def _sc_deg(dst2, zeros16, ones16):
    return pl.kernel(
        _sc_deg_body,
        out_type=jax.ShapeDtypeStruct((NC, NPAD, DEGW), jnp.float32),
        mesh=_sc_mesh(),
        scratch_types=[
            pltpu.VMEM((IDXB, CHUNK), jnp.int32),
            pltpu.VMEM((CHUNK, DEGW), jnp.float32),
            pltpu.VMEM((CHUNK, DEGW), jnp.float32),
            pltpu.VMEM_SHARED((NPAD, DEGW), jnp.float32),
            pltpu.SemaphoreType.DMA,
            pltpu.SemaphoreType.DMA,
        ],
    )(dst2, zeros16, ones16)


def _k0_body(x, w1, h1):
    h1[...] = jnp.dot(x[...], w1[...], preferred_element_type=jnp.float32)


def _k1_body(h1, parts, hs1, dinvb):
    i = pl.program_id(0)
    deg = parts[0, :, 0:1] + parts[1, :, 0:1] + 1.0
    rows = lax.broadcasted_iota(jnp.int32, (BLK, 1), 0) + i * BLK
    dvc = jnp.where(rows < NN, lax.rsqrt(deg), 0.0)
    dv = jnp.broadcast_to(dvc, (BLK, DD))
    hs1[...] = dv * h1[...]
    dinvb[...] = dv


def _k2_body(parts, hs, dinvb, b, w, hs_next):
    dv = dinvb[...]
    t = jnp.maximum(dv * (parts[0] + parts[1] + hs[...]) + b[...], 0.0)
    hs_next[...] = dv * jnp.dot(t, w[...], preferred_element_type=jnp.float32)


def _k3_body(parts, hs, dinvb, b, batch, wfc, bfc, wdec, bdec, w3,
             latent, zr, pooled):
    i = pl.program_id(0)
    dv = dinvb[...]
    t2 = jnp.maximum(dv * (parts[0] + parts[1] + hs[...]) + b[...], 0.0)
    bvec = batch[0]
    iota = lax.broadcasted_iota(jnp.int32, (GG, BLK), 0)
    onehot = jnp.where(bvec == iota, 1.0, 0.0)
    contrib = jnp.dot(onehot, t2, preferred_element_type=jnp.float32)

    @pl.when(i == 0)
    def _():
        pooled[...] = contrib

    @pl.when(i > 0)
    def _():
        pooled[...] = pooled[...] + contrib

    @pl.when(i == NBLK - 1)
    def _():
        lat = jnp.maximum(
            jnp.dot(pooled[...], wfc[...], preferred_element_type=jnp.float32)
            + bfc[...], 0.0)
        z = jnp.maximum(
            jnp.dot(lat, wdec[...], preferred_element_type=jnp.float32)
            + bdec[...], 0.0)
        latent[...] = lat
        zr[...] = jnp.dot(z, w3[...], preferred_element_type=jnp.float32)


def _k5_body(batch, dinvb, zr, zh):
    bvec = batch[0]
    iota = lax.broadcasted_iota(jnp.int32, (GG, BLK), 0)
    onehot = jnp.where(bvec == iota, 1.0, 0.0)
    g = lax.dot_general(onehot, zr[...], (((0,), (0,)), ((), ())),
                        preferred_element_type=jnp.float32)
    zh[...] = dinvb[...] * g


def _k6_body(parts, zh, dinvb, b, recon):
    recon[...] = dinvb[...] * (parts[0] + parts[1] + zh[...]) + b[...]


def _row_spec():
    return pl.BlockSpec((BLK, DD), lambda i: (i, 0))


def _parts_spec():
    return pl.BlockSpec((NC, BLK, DD), lambda i: (0, i, 0))


def _full_spec(shape):
    nd = len(shape)
    return pl.BlockSpec(shape, lambda i: (0,) * nd)


def _batch_spec():
    return pl.BlockSpec((1, 1, BLK), lambda i: (i, 0, 0))


def _k0(xpad, W1):
    return pl.pallas_call(
        _k0_body,
        grid=(NBLK,),
        in_specs=[_row_spec(), _full_spec((DD, DD))],
        out_specs=_row_spec(),
        out_shape=jax.ShapeDtypeStruct((NPAD, DD), jnp.float32),
    )(xpad, W1)


def _k1(h1, parts):
    return pl.pallas_call(
        _k1_body,
        grid=(NBLK,),
        in_specs=[_row_spec(),
                  pl.BlockSpec((NC, BLK, DEGW), lambda i: (0, i, 0))],
        out_specs=[_row_spec(), _row_spec()],
        out_shape=[jax.ShapeDtypeStruct((NPAD, DD), jnp.float32)] * 2,
    )(h1, parts)


def _k2(parts, hs, dinvb, b, W):
    return pl.pallas_call(
        _k2_body,
        grid=(NBLK,),
        in_specs=[_parts_spec(), _row_spec(), _row_spec(),
                  _full_spec((1, DD)), _full_spec((DD, DD))],
        out_specs=_row_spec(),
        out_shape=jax.ShapeDtypeStruct((NPAD, DD), jnp.float32),
    )(parts, hs, dinvb, b, W)


def _k3(parts, hs, dinvb, b2, batch3, Wfc, bfc, Wdec, bdec, W3):
    return pl.pallas_call(
        _k3_body,
        grid=(NBLK,),
        in_specs=[_parts_spec(), _row_spec(), _row_spec(),
                  _full_spec((1, DD)), _batch_spec(),
                  _full_spec((DD, GG)), _full_spec((1, GG)),
                  _full_spec((GG, DD)), _full_spec((1, DD)),
                  _full_spec((DD, DD))],
        out_specs=[pl.BlockSpec((GG, GG), lambda i: (0, 0)),
                   pl.BlockSpec((GG, DD), lambda i: (0, 0))],
        out_shape=[jax.ShapeDtypeStruct((GG, GG), jnp.float32),
                   jax.ShapeDtypeStruct((GG, DD), jnp.float32)],
        scratch_shapes=[pltpu.VMEM((GG, DD), jnp.float32)],
    )(parts, hs, dinvb, b2, batch3, Wfc, bfc, Wdec, bdec, W3)


def _k5(batch3, dinvb, zr):
    return pl.pallas_call(
        _k5_body,
        grid=(NBLK,),
        in_specs=[_batch_spec(), _row_spec(), _full_spec((GG, DD))],
        out_specs=_row_spec(),
        out_shape=jax.ShapeDtypeStruct((NPAD, DD), jnp.float32),
    )(batch3, dinvb, zr)


def _k6(parts, zh, dinvb, b3):
    return pl.pallas_call(
        _k6_body,
        grid=(NBLK,),
        in_specs=[_parts_spec(), _row_spec(), _row_spec(),
                  _full_spec((1, DD))],
        out_specs=_row_spec(),
        out_shape=jax.ShapeDtypeStruct((NPAD, DD), jnp.float32),
    )(parts, zh, dinvb, b3)


def kernel(x, edge_index, batch, W1, b1, W2, b2, Wfc, bfc, Wdec, bdec, W3, b3):
    e = edge_index.shape[1]
    cap = NS * (C0_T + C1_T)
    epad = cap * CHUNK
    assert epad >= e

    ei = edge_index.astype(jnp.int32)
    fill = NN + (jnp.arange(epad - e, dtype=jnp.int32) % (NPAD - NN))
    src3 = jnp.concatenate([ei[0], fill]).reshape(cap, CHUNK)
    dst3 = jnp.concatenate([ei[1], fill]).reshape(cap, CHUNK)
    batch3 = jnp.concatenate(
        [batch.astype(jnp.int32), jnp.full((NPAD - NN,), GG, jnp.int32)]
    ).reshape(NBLK, 1, BLK)
    xpad = jnp.pad(x, ((0, NPAD - NN), (0, 0)))
    zeros = jnp.zeros((CHUNK, DD), jnp.float32)
    zeros16 = jnp.zeros((CHUNK, DEGW), jnp.float32)
    ones16 = jnp.ones((CHUNK, DEGW), jnp.float32)

    b1r = b1.reshape(1, DD)
    b2r = b2.reshape(1, DD)
    b3r = b3.reshape(1, DD)
    bfcr = bfc.reshape(1, GG)
    bdecr = bdec.reshape(1, DD)

    degp = _sc_deg(dst3, zeros16, ones16)
    h1 = _k0(xpad, W1)
    hs1, dinvb = _k1(h1, degp)
    p1 = _sc_scatter(hs1, src3, dst3, zeros)
    hs2 = _k2(p1, hs1, dinvb, b1r, W2)
    p2 = _sc_scatter(hs2, src3, dst3, zeros)
    latent, zr = _k3(p2, hs2, dinvb, b2r, batch3,
                     Wfc, bfcr, Wdec, bdecr, W3)
    zh = _k5(batch3, dinvb, zr)
    p3 = _sc_scatter(zh, src3, dst3, zeros)
    recon = _k6(p3, zh, dinvb, b3r)
    return (recon[:NN], latent)
